# baseline (speedup 1.0000x reference)
import functools

import jax
import jax.numpy as jnp
from jax import lax
from jax.experimental import pallas as pl
from jax.experimental.pallas import tpu as pltpu
from jax.experimental.pallas import tpu_sc as plsc

B = 16384
D = 64
DP = 128
NC = 2
NS = 16
NW = NC * NS
L = 16
N_USER = 1000000
N_TITEM = 100000
UCOLS = (N_USER + DP - 1) // DP
ICOLS = (N_TITEM + DP - 1) // DP
MAXM = B + L
NOUT = B + L

_sc_mesh = plsc.VectorSubcoreMesh(core_axis_name="c", subcore_axis_name="s")


def _splat(x):
    return jnp.full((L,), x, jnp.int32)


@functools.partial(
    pl.kernel,
    mesh=_sc_mesh,
    out_type=[
        jax.ShapeDtypeStruct((NOUT, DP), jnp.float32),
        jax.ShapeDtypeStruct((NOUT, DP), jnp.float32),
    ],
    scratch_types=[
        pltpu.VMEM((B // DP, DP), jnp.int32),
        pltpu.VMEM((MAXM,), jnp.int32),
        pltpu.VMEM((MAXM,), jnp.int32),
        pltpu.VMEM((MAXM,), jnp.int32),
        pltpu.VMEM((272,), jnp.int32),
        pltpu.VMEM((272,), jnp.int32),
        pltpu.VMEM((272,), jnp.int32),
        pltpu.VMEM((272,), jnp.int32),
        pltpu.VMEM((4, D, DP), jnp.float32),
        pltpu.VMEM((DP, DP), jnp.float32),
        pltpu.VMEM((1, DP), jnp.int32),
        pltpu.VMEM((32,), jnp.int32),
        pltpu.SemaphoreType.DMA,
        pltpu.SemaphoreType.DMA,
        pltpu.SemaphoreType.DMA,
        pltpu.SemaphoreType.DMA,
        pltpu.SemaphoreType.DMA,
    ],
    compiler_params=pltpu.CompilerParams(use_tc_tiling_on_sc=True,
                                         needs_layout_passes=False),
)
def _sc_gather(user_hbm, item_hbm, su_hbm, ti_hbm, u_out, i_out,
               idx_v, mc_v, mp_v, sp_v, hist_v, offs_v, run_v, cols_v,
               slab_v, stage_v, bidx_v, tmp_v, sem0, sem1, sem2, sem3, sems):
    slab_sems = (sem0, sem1, sem2, sem3)
    wid = lax.axis_index("s") * NC + lax.axis_index("c")
    iota = lax.iota(jnp.int32, L)
    zero16 = jnp.zeros((L,), jnp.int32)
    one16 = jnp.ones((L,), jnp.int32)
    HUGE = jnp.int32(0x7FFFFFF)

    def one_table(src_idx_hbm, tab_hbm, out_hbm, ncols_total, fill0):
        lo = (wid * ncols_total) // NW
        hi = ((wid + 1) * ncols_total) // NW
        pltpu.sync_copy(src_idx_hbm, idx_v)
        for j in range(272 // L):
            hist_v[pl.ds(j * L, L)] = zero16

        def scan_row(r, cnt):
            for k in range(DP // L):
                u = plsc.load_gather(idx_v, [_splat(r), k * L + iota])
                c = jax.lax.shift_right_logical(u, 7)
                m = (c >= lo) & (c < hi)
                plsc.store_compressed(mc_v.at[pl.ds(cnt, L)], c, mask=m)
                bvec = r * DP + k * L + iota
                packed = (bvec << 7) | (u & 127)
                plsc.store_compressed(mp_v.at[pl.ds(cnt, L)], packed, mask=m)
                cnt = cnt + plsc.all_reduce_population_count(m)[0]
            return cnt
        cnt = lax.fori_loop(0, B // DP, scan_row, jnp.int32(0))
        nchunks = (cnt + L - 1) // L

        def sorted_chunk(j):
            c = mc_v[pl.ds(j * L, L)]
            pk = mp_v[pl.ds(j * L, L)]
            valid = (j * L + iota) < cnt
            key = jnp.where(valid, c, HUGE)
            skey, sval = plsc.sort_key_val(key, pk)
            tmp_v[pl.ds(0, L)] = skey
            prev = plsc.load_gather(tmp_v, [jnp.maximum(iota - 1, 0)])
            nxt = plsc.load_gather(tmp_v, [jnp.minimum(iota + 1, L - 1)])
            seg = (skey != prev) | (iota == 0)
            first = plsc.cummax(jnp.where(seg, iota, 0))
            duprank = iota - first
            svalid = skey != HUGE
            is_last = ((nxt != skey) | (iota == L - 1)) & svalid
            cidx = jnp.where(svalid, skey - lo, 0)
            return skey, sval, svalid, duprank, is_last, cidx

        def hist_chunk(j, _):
            skey, sval, svalid, duprank, is_last, cidx = sorted_chunk(j)
            plsc.addupdate_scatter(hist_v, [cidx], duprank + 1, mask=is_last)
            return _
        lax.fori_loop(0, nchunks, hist_chunk, jnp.int32(0))

        def pfx_chunk(j, carry):
            running, ncol = carry
            h = hist_v[pl.ds(j * L, L)]
            incl = plsc.cumsum(h)
            offs = incl - h + running
            offs_v[pl.ds(j * L, L)] = offs
            run_v[pl.ds(j * L, L)] = offs
            m = h > 0
            plsc.store_compressed(cols_v.at[pl.ds(ncol, L)], lo + j * L + iota,
                                  mask=m)
            return (running + incl[L - 1],
                    ncol + plsc.all_reduce_population_count(m)[0])
        _, ncol = lax.fori_loop(0, 272 // L, pfx_chunk,
                                (jnp.int32(0), jnp.int32(0)))
        cols_v[pl.ds(ncol, L)] = zero16
        ncol4 = (ncol + 3) // 4

        def slab_dma(ci, buf):
            col = cols_v[pl.ds(ci, L)][0]
            off = pl.multiple_of(col * DP, DP)
            return pltpu.async_copy(
                tab_hbm.at[:, pl.ds(off, DP)], slab_v.at[buf], slab_sems[buf])

        @pl.when(ncol > 0)
        def _():
            for k in range(4):
                slab_dma(k, k)

        def place_chunk(j, _):
            skey, sval, svalid, duprank, is_last, cidx = sorted_chunk(j)
            base = plsc.load_gather(run_v, [cidx])
            pos = base + duprank
            plsc.store_scatter(sp_v, [pos], sval, mask=svalid)
            plsc.store_scatter(run_v, [cidx], pos + 1, mask=is_last)
            return _
        lax.fori_loop(0, nchunks, place_chunk, jnp.int32(0))

        def per_group(g, fill):
            for k in range(4):
                ci = g * 4 + k
                pltpu.make_async_copy(
                    tab_hbm.at[:, pl.ds(0, DP)], slab_v.at[k],
                    slab_sems[k]).wait()

                @pl.when(ci < ncol)
                def _():
                    pass
                col = cols_v[pl.ds(ci, L)][0]
                colc = jnp.where(ci < ncol, col - lo, 0)
                start = plsc.load_gather(offs_v, [_splat(colc)])[0]
                num = plsc.load_gather(hist_v, [_splat(colc)])[0]
                num = jnp.where(ci < ncol, num, 0)

                def per_match(t, fill):
                    pk = plsc.load_gather(sp_v, [_splat(start + t)])
                    lane = pk & 127
                    b = jax.lax.shift_right_logical(pk, 7)
                    slot = fill & (DP - 1)
                    for q in range(D // L):
                        gv = plsc.load_gather(slab_v.at[k],
                                              [q * L + iota, lane])
                        plsc.store_scatter(stage_v,
                                           [_splat(slot), q * L + iota], gv)
                    plsc.store_scatter(bidx_v, [zero16, _splat(slot)], b,
                                       mask=(iota == 0))
                    fill = fill + 1

                    @pl.when((fill & (DP - 1)) == 0)
                    def _():
                        pltpu.async_copy(
                            stage_v, out_hbm.at[bidx_v.at[0]], sems).wait()
                    return fill
                fill = lax.fori_loop(0, num, per_match, fill)
                ri = (g + 1) * 4 + k

                @pl.when(ri < ncol4 * 4)
                def _():
                    slab_dma(ri, k)
            return fill

        fill = lax.fori_loop(0, ncol4, per_group, fill0)

        rem = fill & (DP - 1)

        @pl.when(rem > 0)
        def _():
            def pad_slot(s, _):
                plsc.store_scatter(bidx_v, [zero16, _splat(s)],
                                   _splat(B) + (_splat(s) & 15),
                                   mask=(iota == 0))
                return _
            lax.fori_loop(rem, DP, pad_slot, jnp.int32(0))
            pltpu.async_copy(stage_v, out_hbm.at[bidx_v.at[0]], sems).wait()

    one_table(user_hbm, su_hbm, u_out, UCOLS, jnp.int32(0))
    one_table(item_hbm, ti_hbm, i_out, ICOLS, jnp.int32(0))


BLK = 8192


def _mlp_body(u_ref, i_ref, w1_ref, b1_ref, w2_ref, b2_ref,
              w3_ref, b3_ref, w4_ref, b4_ref, out_ref):
    u = u_ref[:, :D]
    iv = i_ref[:, :D]
    h = jnp.maximum(jnp.dot(u, w1_ref[...], preferred_element_type=jnp.float32)
                    + b1_ref[...], 0.0)
    h = jnp.maximum(jnp.dot(h, w2_ref[...], preferred_element_type=jnp.float32)
                    + b2_ref[...], 0.0)
    h = jnp.maximum(jnp.dot(h, w3_ref[...], preferred_element_type=jnp.float32)
                    + b3_ref[...], 0.0)
    fu = jnp.dot(h, w4_ref[...], preferred_element_type=jnp.float32) + b4_ref[...]
    out_ref[0, 0, :] = jnp.sum(fu * iv, axis=1)


def _tc_mlp(u_rows, i_rows, w1t, b1, w2t, b2, w3t, b3, w4t, b4):
    grid = B // BLK
    return pl.pallas_call(
        _mlp_body,
        grid=(grid,),
        in_specs=[
            pl.BlockSpec((BLK, DP), lambda i: (i, 0)),
            pl.BlockSpec((BLK, DP), lambda i: (i, 0)),
            pl.BlockSpec(w1t.shape, lambda i: (0, 0)),
            pl.BlockSpec(b1.shape, lambda i: (0, 0)),
            pl.BlockSpec(w2t.shape, lambda i: (0, 0)),
            pl.BlockSpec(b2.shape, lambda i: (0, 0)),
            pl.BlockSpec(w3t.shape, lambda i: (0, 0)),
            pl.BlockSpec(b3.shape, lambda i: (0, 0)),
            pl.BlockSpec(w4t.shape, lambda i: (0, 0)),
            pl.BlockSpec(b4.shape, lambda i: (0, 0)),
        ],
        out_specs=pl.BlockSpec((1, 1, BLK), lambda i: (i, 0, 0)),
        out_shape=jax.ShapeDtypeStruct((grid, 1, BLK), jnp.float32),
    )(u_rows, i_rows, w1t, b1, w2t, b2, w3t, b3, w4t, b4)


def kernel(user, item, su_emb, ti_emb, W1, b1, W2, b2, W3, b3, W4, b4):
    user = user.astype(jnp.int32)
    item = item.astype(jnp.int32)
    u_rows, i_rows = _sc_gather(user.reshape(B // DP, DP),
                                item.reshape(B // DP, DP),
                                su_emb.T, ti_emb.T)
    score = _tc_mlp(u_rows, i_rows,
                    W1.T, b1.reshape(1, -1),
                    W2.T, b2.reshape(1, -1),
                    W3.T, b3.reshape(1, -1),
                    W4.T, b4.reshape(1, -1))
    return score.reshape(B)


# speedup vs baseline: 1.0961x; 1.0961x over previous
import functools

import jax
import jax.numpy as jnp
from jax import lax
from jax.experimental import pallas as pl
from jax.experimental.pallas import tpu as pltpu
from jax.experimental.pallas import tpu_sc as plsc

B = 16384
D = 64
DP = 128
NC = 2
NS = 16
NW = NC * NS
L = 16
N_USER = 1000000
N_TITEM = 100000
UCOLS = (N_USER + DP - 1) // DP
ICOLS = (N_TITEM + DP - 1) // DP
MAXM = B + L
NOUT = B + L

_sc_mesh = plsc.VectorSubcoreMesh(core_axis_name="c", subcore_axis_name="s")


def _splat(x):
    return jnp.full((L,), x, jnp.int32)


@functools.partial(
    pl.kernel,
    mesh=_sc_mesh,
    out_type=[
        jax.ShapeDtypeStruct((NOUT, DP), jnp.float32),
        jax.ShapeDtypeStruct((NOUT, DP), jnp.float32),
    ],
    scratch_types=[
        pltpu.VMEM((B // DP, DP), jnp.int32),
        pltpu.VMEM((MAXM,), jnp.int32),
        pltpu.VMEM((MAXM,), jnp.int32),
        pltpu.VMEM((MAXM,), jnp.int32),
        pltpu.VMEM((272,), jnp.int32),
        pltpu.VMEM((272,), jnp.int32),
        pltpu.VMEM((272,), jnp.int32),
        pltpu.VMEM((272,), jnp.int32),
        pltpu.VMEM((4, D, DP), jnp.float32),
        pltpu.VMEM((DP, DP), jnp.float32),
        pltpu.VMEM((1, DP), jnp.int32),
        pltpu.VMEM((32,), jnp.int32),
        pltpu.SemaphoreType.DMA,
        pltpu.SemaphoreType.DMA,
        pltpu.SemaphoreType.DMA,
        pltpu.SemaphoreType.DMA,
        pltpu.SemaphoreType.DMA,
    ],
    compiler_params=pltpu.CompilerParams(use_tc_tiling_on_sc=True,
                                         needs_layout_passes=False),
)
def _sc_gather(user_hbm, item_hbm, su_hbm, ti_hbm, u_out, i_out,
               idx_v, mc_v, mp_v, sp_v, hist_v, offs_v, run_v, cols_v,
               slab_v, stage_v, bidx_v, tmp_v, sem0, sem1, sem2, sem3, sems):
    slab_sems = (sem0, sem1, sem2, sem3)
    wid = lax.axis_index("s") * NC + lax.axis_index("c")
    iota = lax.iota(jnp.int32, L)
    zero16 = jnp.zeros((L,), jnp.int32)
    one16 = jnp.ones((L,), jnp.int32)
    HUGE = jnp.int32(0x7FFFFFF)

    def one_table(src_idx_hbm, tab_hbm, out_hbm, ncols_total, fill0):
        lo = (wid * ncols_total) // NW
        hi = ((wid + 1) * ncols_total) // NW
        pltpu.sync_copy(src_idx_hbm, idx_v)
        for j in range(272 // L):
            hist_v[pl.ds(j * L, L)] = zero16

        def scan_row(r, cnt):
            for k in range(DP // L):
                u = plsc.load_gather(idx_v, [_splat(r), k * L + iota])
                c = jax.lax.shift_right_logical(u, 7)
                m = (c >= lo) & (c < hi)
                plsc.store_compressed(mc_v.at[pl.ds(cnt, L)], c, mask=m)
                bvec = r * DP + k * L + iota
                packed = (bvec << 7) | (u & 127)
                plsc.store_compressed(mp_v.at[pl.ds(cnt, L)], packed, mask=m)
                cnt = cnt + plsc.all_reduce_population_count(m)[0]
            return cnt
        cnt = lax.fori_loop(0, B // DP, scan_row, jnp.int32(0))
        nchunks = (cnt + L - 1) // L

        def sorted_chunk(j):
            c = mc_v[pl.ds(j * L, L)]
            pk = mp_v[pl.ds(j * L, L)]
            valid = (j * L + iota) < cnt
            key = jnp.where(valid, c, HUGE)
            skey, sval = plsc.sort_key_val(key, pk)
            tmp_v[pl.ds(0, L)] = skey
            prev = plsc.load_gather(tmp_v, [jnp.maximum(iota - 1, 0)])
            nxt = plsc.load_gather(tmp_v, [jnp.minimum(iota + 1, L - 1)])
            seg = (skey != prev) | (iota == 0)
            first = plsc.cummax(jnp.where(seg, iota, 0))
            duprank = iota - first
            svalid = skey != HUGE
            is_last = ((nxt != skey) | (iota == L - 1)) & svalid
            cidx = jnp.where(svalid, skey - lo, 0)
            return skey, sval, svalid, duprank, is_last, cidx

        def hist_chunk(j, _):
            skey, sval, svalid, duprank, is_last, cidx = sorted_chunk(j)
            plsc.addupdate_scatter(hist_v, [cidx], duprank + 1, mask=is_last)
            return _
        lax.fori_loop(0, nchunks, hist_chunk, jnp.int32(0))

        def pfx_chunk(j, carry):
            running, ncol = carry
            h = hist_v[pl.ds(j * L, L)]
            incl = plsc.cumsum(h)
            offs = incl - h + running
            offs_v[pl.ds(j * L, L)] = offs
            run_v[pl.ds(j * L, L)] = offs
            m = h > 0
            plsc.store_compressed(cols_v.at[pl.ds(ncol, L)], lo + j * L + iota,
                                  mask=m)
            return (running + incl[L - 1],
                    ncol + plsc.all_reduce_population_count(m)[0])
        _, ncol = lax.fori_loop(0, 272 // L, pfx_chunk,
                                (jnp.int32(0), jnp.int32(0)))
        cols_v[pl.ds(ncol, L)] = zero16
        ncol4 = (ncol + 3) // 4

        def slab_dma(ci, buf):
            col = cols_v[pl.ds(ci, L)][0]
            off = pl.multiple_of(col * DP, DP)
            return pltpu.async_copy(
                tab_hbm.at[:, pl.ds(off, DP)], slab_v.at[buf], slab_sems[buf])

        @pl.when(ncol > 0)
        def _():
            for k in range(4):
                slab_dma(k, k)

        def place_chunk(j, _):
            skey, sval, svalid, duprank, is_last, cidx = sorted_chunk(j)
            base = plsc.load_gather(run_v, [cidx])
            pos = base + duprank
            plsc.store_scatter(sp_v, [pos], sval, mask=svalid)
            plsc.store_scatter(run_v, [cidx], pos + 1, mask=is_last)
            return _
        lax.fori_loop(0, nchunks, place_chunk, jnp.int32(0))

        def per_group(g, fill):
            for k in range(4):
                ci = g * 4 + k
                pltpu.make_async_copy(
                    tab_hbm.at[:, pl.ds(0, DP)], slab_v.at[k],
                    slab_sems[k]).wait()

                @pl.when(ci < ncol)
                def _():
                    pass
                col = cols_v[pl.ds(ci, L)][0]
                colc = jnp.where(ci < ncol, col - lo, 0)
                start = plsc.load_gather(offs_v, [_splat(colc)])[0]
                num = plsc.load_gather(hist_v, [_splat(colc)])[0]
                num = jnp.where(ci < ncol, num, 0)

                def per_match(t, fill):
                    pk = plsc.load_gather(sp_v, [_splat(start + t)])
                    lane = pk & 127
                    b = jax.lax.shift_right_logical(pk, 7)
                    slot = fill & (DP - 1)
                    for q in range(D // L):
                        gv = plsc.load_gather(slab_v.at[k],
                                              [q * L + iota, lane])
                        plsc.store_scatter(stage_v,
                                           [_splat(slot), q * L + iota], gv)
                    plsc.store_scatter(bidx_v, [zero16, _splat(slot)], b,
                                       mask=(iota == 0))
                    fill = fill + 1

                    @pl.when((fill & (DP - 1)) == 0)
                    def _():
                        pltpu.async_copy(
                            stage_v, out_hbm.at[bidx_v.at[0]], sems).wait()
                    return fill
                fill = lax.fori_loop(0, num, per_match, fill)
                ri = (g + 1) * 4 + k

                @pl.when(ri < ncol4 * 4)
                def _():
                    slab_dma(ri, k)
            return fill

        fill = lax.fori_loop(0, ncol4, per_group, fill0)

        rem = fill & (DP - 1)

        @pl.when(rem > 0)
        def _():
            def pad_slot(s, _):
                plsc.store_scatter(bidx_v, [zero16, _splat(s)],
                                   _splat(B) + (_splat(s) & 15),
                                   mask=(iota == 0))
                return _
            lax.fori_loop(rem, DP, pad_slot, jnp.int32(0))
            pltpu.async_copy(stage_v, out_hbm.at[bidx_v.at[0]], sems).wait()

    one_table(user_hbm, su_hbm, u_out, UCOLS, jnp.int32(0))
    one_table(item_hbm, ti_hbm, i_out, ICOLS, jnp.int32(0))


BLK = 4096


def _mlp_body(u_ref, i_ref, w1_ref, b1_ref, w2_ref, b2_ref,
              w3_ref, b3_ref, w4_ref, b4_ref, out_ref):
    u = u_ref[:, :D]
    iv = i_ref[:, :D]
    h = jnp.maximum(jnp.dot(u, w1_ref[...], preferred_element_type=jnp.float32)
                    + b1_ref[...], 0.0)
    h = jnp.maximum(jnp.dot(h, w2_ref[...], preferred_element_type=jnp.float32)
                    + b2_ref[...], 0.0)
    h = jnp.maximum(jnp.dot(h, w3_ref[...], preferred_element_type=jnp.float32)
                    + b3_ref[...], 0.0)
    fu = jnp.dot(h, w4_ref[...], preferred_element_type=jnp.float32) + b4_ref[...]
    ones = jnp.ones((D, 1), jnp.float32)
    out_ref[0, 0, :] = jnp.dot(fu * iv, ones,
                               preferred_element_type=jnp.float32)[:, 0]


def _tc_mlp(u_rows, i_rows, w1t, b1, w2t, b2, w3t, b3, w4t, b4):
    grid = B // BLK
    return pl.pallas_call(
        _mlp_body,
        grid=(grid,),
        in_specs=[
            pl.BlockSpec((BLK, DP), lambda i: (i, 0)),
            pl.BlockSpec((BLK, DP), lambda i: (i, 0)),
            pl.BlockSpec(w1t.shape, lambda i: (0, 0)),
            pl.BlockSpec(b1.shape, lambda i: (0, 0)),
            pl.BlockSpec(w2t.shape, lambda i: (0, 0)),
            pl.BlockSpec(b2.shape, lambda i: (0, 0)),
            pl.BlockSpec(w3t.shape, lambda i: (0, 0)),
            pl.BlockSpec(b3.shape, lambda i: (0, 0)),
            pl.BlockSpec(w4t.shape, lambda i: (0, 0)),
            pl.BlockSpec(b4.shape, lambda i: (0, 0)),
        ],
        out_specs=pl.BlockSpec((1, 1, BLK), lambda i: (i, 0, 0)),
        out_shape=jax.ShapeDtypeStruct((grid, 1, BLK), jnp.float32),
    )(u_rows, i_rows, w1t, b1, w2t, b2, w3t, b3, w4t, b4)


def kernel(user, item, su_emb, ti_emb, W1, b1, W2, b2, W3, b3, W4, b4):
    user = user.astype(jnp.int32)
    item = item.astype(jnp.int32)
    u_rows, i_rows = _sc_gather(user.reshape(B // DP, DP),
                                item.reshape(B // DP, DP),
                                su_emb.T, ti_emb.T)
    score = _tc_mlp(u_rows, i_rows,
                    W1.T, b1.reshape(1, -1),
                    W2.T, b2.reshape(1, -1),
                    W3.T, b3.reshape(1, -1),
                    W4.T, b4.reshape(1, -1))
    return score.reshape(B)
